# baseline (speedup 1.0000x reference)
import functools

import jax
import jax.numpy as jnp
from jax import lax
from jax.experimental import pallas as pl
from jax.experimental.pallas import tpu as pltpu
from jax.experimental.pallas import tpu_sc as plsc

_B = 32
_ROW = 160000
_NCH = 10
_CHUNK = _ROW // _NCH
_TOT = 2 * _B * _NCH
_NW = 32
_RPW = _TOT // _NW
_K = 4
_NG = _RPW // _K


def _make_remix():
    mesh = plsc.VectorSubcoreMesh(core_axis_name="c", subcore_axis_name="s")

    @functools.partial(
        pl.kernel,
        out_type=jax.ShapeDtypeStruct((_TOT, _CHUNK), jnp.float32),
        mesh=mesh,
        scratch_types=[
            pltpu.VMEM((_NG, _K), jnp.int32),
            pltpu.VMEM((_K, _CHUNK), jnp.float32),
            pltpu.SemaphoreType.DMA,
            pltpu.SemaphoreType.DMA,
        ],
    )
    def remix(src_hbm, idx_hbm, out_hbm, idx_v, buf, gsem, ssem):
        wid = lax.axis_index("s") * 2 + lax.axis_index("c")
        base = wid * _RPW
        pltpu.sync_copy(idx_hbm.at[wid], idx_v)
        for g in range(_NG):
            pltpu.async_copy(
                src_hbm.at[idx_v.at[g]], buf, gsem
            ).wait()
            pltpu.async_copy(
                buf, out_hbm.at[pl.ds(base + g * _K, _K)], ssem
            ).wait()

    return remix


_remix = _make_remix()


def kernel(sources):
    perm = jnp.argsort(jax.random.uniform(jax.random.key(42), (_B,)))
    idx_noise = (perm[:, None] * _NCH + jnp.arange(_NCH)[None, :]).reshape(-1)
    idx_clean = jnp.arange(_B * _NCH, 2 * _B * _NCH)
    idx_all = (
        jnp.concatenate([idx_noise, idx_clean])
        .astype(jnp.int32)
        .reshape(_NW, _NG, _K)
    )
    src = sources.reshape(_TOT, _CHUNK)
    out = _remix(src, idx_all)
    return out.reshape(2, _B, 1, _ROW)


# speedup vs baseline: 1.0128x; 1.0128x over previous
import functools

import jax
import jax.numpy as jnp
from jax import lax
from jax.experimental import pallas as pl
from jax.experimental.pallas import tpu as pltpu
from jax.experimental.pallas import tpu_sc as plsc

_B = 32
_ROW = 160000
_NCH = 10
_CHUNK = _ROW // _NCH
_TOT = 2 * _B * _NCH
_NW = 32
_RPW = _TOT // _NW
_K = 4
_NG = _RPW // _K


def _make_remix():
    mesh = plsc.VectorSubcoreMesh(core_axis_name="c", subcore_axis_name="s")

    @functools.partial(
        pl.kernel,
        out_type=jax.ShapeDtypeStruct((_TOT, _CHUNK), jnp.float32),
        mesh=mesh,
        scratch_types=[
            pltpu.VMEM((_NG, _K), jnp.int32),
            pltpu.VMEM((_K, _CHUNK), jnp.float32),
            pltpu.VMEM((_K, _CHUNK), jnp.float32),
            pltpu.SemaphoreType.DMA,
            pltpu.SemaphoreType.DMA,
            pltpu.SemaphoreType.DMA,
            pltpu.SemaphoreType.DMA,
        ],
    )
    def remix(src_hbm, idx_hbm, out_hbm, idx_v, buf0, buf1, g0, g1, s0, s1):
        wid = lax.axis_index("s") * 2 + lax.axis_index("c")
        base = wid * _RPW
        pltpu.sync_copy(idx_hbm.at[wid], idx_v)
        bufs, gsem, ssem = [buf0, buf1], [g0, g1], [s0, s1]
        gh = [None] * _NG
        sh = [None] * _NG
        for g in range(_NG):
            if g >= 2:
                sh[g - 2].wait()
            gh[g] = pltpu.async_copy(
                src_hbm.at[idx_v.at[g]], bufs[g % 2], gsem[g % 2]
            )
            if g >= 1:
                gh[g - 1].wait()
                sh[g - 1] = pltpu.async_copy(
                    bufs[(g - 1) % 2],
                    out_hbm.at[pl.ds(base + (g - 1) * _K, _K)],
                    ssem[(g - 1) % 2],
                )
        gh[_NG - 1].wait()
        sh[_NG - 1] = pltpu.async_copy(
            bufs[(_NG - 1) % 2],
            out_hbm.at[pl.ds(base + (_NG - 1) * _K, _K)],
            ssem[(_NG - 1) % 2],
        )
        sh[_NG - 2].wait()
        sh[_NG - 1].wait()

    return remix


_remix = _make_remix()


def kernel(sources):
    perm = jnp.argsort(jax.random.uniform(jax.random.key(42), (_B,)))
    idx_noise = (perm[:, None] * _NCH + jnp.arange(_NCH)[None, :]).reshape(-1)
    idx_clean = jnp.arange(_B * _NCH, 2 * _B * _NCH)
    idx_all = (
        jnp.concatenate([idx_noise, idx_clean])
        .astype(jnp.int32)
        .reshape(_NW, _NG, _K)
    )
    src = sources.reshape(_TOT, _CHUNK)
    out = _remix(src, idx_all)
    return out.reshape(2, _B, 1, _ROW)


# speedup vs baseline: 6.3568x; 6.2761x over previous
import functools

import jax
import jax.numpy as jnp
from jax import lax
from jax.experimental import pallas as pl
from jax.experimental.pallas import tpu as pltpu
from jax.experimental.pallas import tpu_sc as plsc

_B = 32
_ROW = 160000
_NCH = 10
_CHUNK = _ROW // _NCH
_TOT = 2 * _B * _NCH
_NW = 32
_RPW = _TOT // _NW
_K = 4
_NG = _RPW // _K


def _make_remix():
    mesh = plsc.VectorSubcoreMesh(core_axis_name="c", subcore_axis_name="s")

    @functools.partial(
        pl.kernel,
        out_type=jax.ShapeDtypeStruct((_TOT, 1, _CHUNK), jnp.float32),
        mesh=mesh,
        scratch_types=[
            pltpu.VMEM((_NG, _K), jnp.int32),
            pltpu.VMEM((_K, 1, _CHUNK), jnp.float32),
            pltpu.VMEM((_K, 1, _CHUNK), jnp.float32),
            pltpu.SemaphoreType.DMA,
            pltpu.SemaphoreType.DMA,
            pltpu.SemaphoreType.DMA,
            pltpu.SemaphoreType.DMA,
        ],
    )
    def remix(src_hbm, idx_hbm, out_hbm, idx_v, buf0, buf1, g0, g1, s0, s1):
        wid = lax.axis_index("s") * 2 + lax.axis_index("c")
        base = wid * _RPW
        pltpu.sync_copy(idx_hbm.at[wid], idx_v)
        bufs, gsem, ssem = [buf0, buf1], [g0, g1], [s0, s1]
        gh = [None] * _NG
        sh = [None] * _NG
        for g in range(_NG):
            if g >= 2:
                sh[g - 2].wait()
            gh[g] = pltpu.async_copy(
                src_hbm.at[idx_v.at[g]], bufs[g % 2], gsem[g % 2]
            )
            if g >= 1:
                gh[g - 1].wait()
                sh[g - 1] = pltpu.async_copy(
                    bufs[(g - 1) % 2],
                    out_hbm.at[pl.ds(base + (g - 1) * _K, _K)],
                    ssem[(g - 1) % 2],
                )
        gh[_NG - 1].wait()
        sh[_NG - 1] = pltpu.async_copy(
            bufs[(_NG - 1) % 2],
            out_hbm.at[pl.ds(base + (_NG - 1) * _K, _K)],
            ssem[(_NG - 1) % 2],
        )
        sh[_NG - 2].wait()
        sh[_NG - 1].wait()

    return remix


_remix = _make_remix()


def kernel(sources):
    perm = jnp.argsort(jax.random.uniform(jax.random.key(42), (_B,)))
    idx_noise = (perm[:, None] * _NCH + jnp.arange(_NCH)[None, :]).reshape(-1)
    idx_clean = jnp.arange(_B * _NCH, 2 * _B * _NCH)
    idx_all = (
        jnp.concatenate([idx_noise, idx_clean])
        .astype(jnp.int32)
        .reshape(_NW, _NG, _K)
    )
    src = sources.reshape(_TOT, 1, _CHUNK)
    out = _remix(src, idx_all)
    return out.reshape(2, _B, 1, _ROW)
